# baseline (speedup 1.0000x reference)
import functools

import jax
import jax.numpy as jnp
from jax import lax
from jax.experimental import pallas as pl
from jax.experimental.pallas import tpu as pltpu
from jax.experimental.pallas import tpu_sc as plsc

_LANES = 16
_NC = 2
_NS = 16
_CHUNK = 40


def _tc_scaling_body(ea_ref, w_ref, b_ref, out_ref, *, icp):
    s = jnp.dot(ea_ref[...], w_ref[...], preferred_element_type=jnp.float32)
    s = jnp.maximum(s + b_ref[...], 0.0)
    out_ref[0] = s[:, :icp]
    out_ref[1] = s[:, icp:]


def _tc_out_body(a0_ref, a1_ref, w0_ref, w1_ref, b_ref, out_ref):
    acc = jnp.dot(a0_ref[...], w0_ref[...], preferred_element_type=jnp.float32)
    acc = acc + jnp.dot(a1_ref[...], w1_ref[...], preferred_element_type=jnp.float32)
    out_ref[...] = jnp.tanh(acc + b_ref[...])


def _sc_gather_scale_scatter(e, icp, npad):
    ept = e // _NS
    nchunk = ept // _CHUNK
    rpt = npad // _NS
    nvec = icp // _LANES
    zrows = 16

    npairs = nchunk // 2

    mesh = plsc.VectorSubcoreMesh(
        core_axis_name="c", subcore_axis_name="s",
        num_cores=_NC, num_subcores=_NS)

    @functools.partial(
        pl.kernel,
        out_type=jax.ShapeDtypeStruct((_NC * npad, icp), jnp.float32),
        mesh=mesh,
        scratch_types=[
            pltpu.VMEM((_CHUNK,), jnp.int32),
            pltpu.VMEM((_CHUNK,), jnp.int32),
            pltpu.VMEM((_CHUNK,), jnp.int32),
            pltpu.VMEM((_CHUNK,), jnp.int32),
            pltpu.VMEM((_CHUNK, icp), jnp.float32),
            pltpu.VMEM((_CHUNK, icp), jnp.float32),
            pltpu.VMEM((_CHUNK, icp), jnp.float32),
            pltpu.VMEM((_CHUNK, icp), jnp.float32),
            pltpu.VMEM_SHARED((npad, icp), jnp.float32),
            pltpu.VMEM((zrows, icp), jnp.float32),
            pltpu.SemaphoreType.DMA,
            pltpu.SemaphoreType.DMA,
            pltpu.SemaphoreType.DMA,
            pltpu.SemaphoreType.DMA,
        ],
        compiler_params=pltpu.CompilerParams(use_tc_tiling_on_sc=False),
    )
    def sc_kernel(xc_hbm, src_hbm, dst_hbm, s_hbm, out_hbm,
                  srcv0, srcv1, dstv0, dstv1, sv0, sv1, xv0, xv1,
                  acc, zbuf, ls0, ls1, gs0, gs1):
        c = lax.axis_index("c")
        s = lax.axis_index("s")
        srcv = (srcv0, srcv1)
        dstv = (dstv0, dstv1)
        sv = (sv0, sv1)
        xv = (xv0, xv1)
        ls = (ls0, ls1)
        gs = (gs0, gs1)

        def zrow(r, _):
            for k in range(nvec):
                zbuf[r, pl.ds(k * _LANES, _LANES)] = jnp.zeros(
                    (_LANES,), jnp.float32)
            return 0
        lax.fori_loop(0, zrows, zrow, 0)

        def zcopy(j, _):
            pltpu.sync_copy(
                zbuf, acc.at[pl.ds(s * rpt + j * zrows, zrows), :])
            return 0
        lax.fori_loop(0, rpt // zrows, zcopy, 0)
        plsc.subcore_barrier()

        def loads_descr(cj, b):
            base = s * ept + cj * _CHUNK
            return (
                pltpu.make_async_copy(
                    src_hbm.at[pl.ds(base, _CHUNK)], srcv[b], ls[b]),
                pltpu.make_async_copy(
                    dst_hbm.at[pl.ds(base, _CHUNK)], dstv[b], ls[b]),
                pltpu.make_async_copy(
                    s_hbm.at[pl.ds(c * e + base, _CHUNK), :], sv[b], ls[b]),
            )

        def issue_loads(cj, b):
            for d in loads_descr(cj, b):
                d.start()

        def wait_loads(cj, b):
            for d in loads_descr(cj, b):
                d.wait()

        def gather_descr(b):
            return pltpu.make_async_copy(xc_hbm.at[srcv[b]], xv[b], gs[b])

        def process(cj, b, nb):
            cj1 = jnp.minimum(cj + 1, nchunk - 1)
            wait_loads(cj1, nb)
            gather_descr(nb).start()
            gather_descr(b).wait()

            for i in range(_CHUNK):
                for k in range(nvec):
                    sl = pl.ds(k * _LANES, _LANES)
                    sv[b][i, sl] = sv[b][i, sl] * xv[b][i, sl]

            pltpu.sync_copy(sv[b], acc.at[dstv[b]], add=True)

            cj2 = jnp.minimum(cj + 2, nchunk - 1)
            issue_loads(cj2, b)

        issue_loads(0, 0)
        wait_loads(0, 0)
        gather_descr(0).start()
        issue_loads(1, 1)
        def pair(j, _):
            process(2 * j, 0, 1)
            process(2 * j + 1, 1, 0)
            return 0
        lax.fori_loop(0, npairs, pair, 0)
        gather_descr(0).wait()
        wait_loads(nchunk - 1, 1)
        plsc.subcore_barrier()

        pltpu.sync_copy(
            acc.at[pl.ds(s * rpt, rpt), :],
            out_hbm.at[pl.ds(c * npad + s * rpt, rpt), :])

    return sc_kernel


def kernel(x, edge_index, edge_attr, node_attr, W_in, b_in, W_out, b_out):
    n, d = x.shape
    na = node_attr.shape[1]
    e = edge_index.shape[1]
    ea = edge_attr.shape[1]
    ic = d + na
    oc = W_out.shape[0]
    icp = ((ic + _LANES - 1) // _LANES) * _LANES
    npad = ((n + _NS * 16 - 1) // (_NS * 16)) * (_NS * 16)

    f32 = jnp.float32
    src = edge_index[0].astype(jnp.int32)
    dst = edge_index[1].astype(jnp.int32)

    xc = jnp.concatenate([x.astype(f32), node_attr.astype(f32)], axis=1)
    xcp = jnp.pad(xc, ((0, 0), (0, icp - ic)))

    w_h = [jnp.pad(W_in[h::2, :], ((0, icp - ic), (0, 0))) for h in range(2)]
    w_cat = jnp.concatenate(w_h, axis=0).T.astype(f32)
    b_h = [jnp.pad(b_in[h::2], (0, icp - ic)) for h in range(2)]
    b_cat = jnp.concatenate(b_h, axis=0)[None, :].astype(f32)

    tile_e = 640
    scal = pl.pallas_call(
        functools.partial(_tc_scaling_body, icp=icp),
        grid=(e // tile_e,),
        in_specs=[
            pl.BlockSpec((tile_e, ea), lambda i: (i, 0)),
            pl.BlockSpec((ea, 2 * icp), lambda i: (0, 0)),
            pl.BlockSpec((1, 2 * icp), lambda i: (0, 0)),
        ],
        out_specs=pl.BlockSpec((2, tile_e, icp), lambda i: (0, i, 0)),
        out_shape=jax.ShapeDtypeStruct((2, e, icp), f32),
    )(edge_attr.astype(f32), w_cat, b_cat)
    scal_flat = scal.reshape(2 * e, icp)

    sc_fn = _sc_gather_scale_scatter(e, icp, npad)
    aggr = sc_fn(xcp, src, dst, scal_flat)
    a0 = aggr[:n]
    a1 = aggr[npad:npad + n]

    w0o = jnp.pad(W_out[:, 0::2], ((0, 0), (0, icp - ic))).T.astype(f32)
    w1o = jnp.pad(W_out[:, 1::2], ((0, 0), (0, icp - ic))).T.astype(f32)
    b_o = b_out[None, :].astype(f32)

    tile_n = 1000
    out = pl.pallas_call(
        _tc_out_body,
        grid=(n // tile_n,),
        in_specs=[
            pl.BlockSpec((tile_n, icp), lambda i: (i, 0)),
            pl.BlockSpec((tile_n, icp), lambda i: (i, 0)),
            pl.BlockSpec((icp, oc), lambda i: (0, 0)),
            pl.BlockSpec((icp, oc), lambda i: (0, 0)),
            pl.BlockSpec((1, oc), lambda i: (0, 0)),
        ],
        out_specs=pl.BlockSpec((tile_n, oc), lambda i: (i, 0)),
        out_shape=jax.ShapeDtypeStruct((n, oc), f32),
    )(a0, a1, w0o, w1o, b_o)
    return out


# speedup vs baseline: 1.5189x; 1.5189x over previous
import functools

import jax
import jax.numpy as jnp
from jax import lax
from jax.experimental import pallas as pl
from jax.experimental.pallas import tpu as pltpu
from jax.experimental.pallas import tpu_sc as plsc

_LANES = 16
_NC = 2
_NS = 16
_CHUNK = 64
_GRP = 8
_ROWS = 9


def _tc_scaling_body(ea8_ref, w_ref, b_ref, out_ref, *, tt):
    ea8 = ea8_ref[...]
    w = w_ref[0]
    b = b_ref[0]
    for L in range(_ROWS):
        m = jnp.dot(ea8, w[:, 128 * L:128 * (L + 1)],
                    preferred_element_type=jnp.float32)
        m = jnp.maximum(m + b[:, 128 * L:128 * (L + 1)], 0.0)
        out_ref[0, :, 8 * L:8 * (L + 1), :] = m.reshape(tt // 8, 8, 128)


def _tc_out_body(a0_ref, a1_ref, w0_ref, w1_ref, b_ref, out_ref):
    acc = jnp.dot(a0_ref[...], w0_ref[...], preferred_element_type=jnp.float32)
    acc = acc + jnp.dot(a1_ref[...], w1_ref[...], preferred_element_type=jnp.float32)
    out_ref[...] = jnp.tanh(acc + b_ref[...])


def _sc_gather_scale_scatter(ntr, icp, npad):
    trpt = ntr // _NS
    npairs = trpt // 2
    rpt = npad // _NS
    nvec = icp // _LANES
    zrows = 8
    srows = _ROWS * _GRP

    dk = [(d, k, 8 * ((144 * d + 16 * k) // 128), (144 * d + 16 * k) % 128)
          for d in range(_GRP) for k in range(nvec)]

    mesh = plsc.VectorSubcoreMesh(
        core_axis_name="c", subcore_axis_name="s",
        num_cores=_NC, num_subcores=_NS)

    @functools.partial(
        pl.kernel,
        out_type=jax.ShapeDtypeStruct((_NC * npad, icp), jnp.float32),
        mesh=mesh,
        scratch_types=[
            pltpu.VMEM((_CHUNK,), jnp.int32),
            pltpu.VMEM((_CHUNK,), jnp.int32),
            pltpu.VMEM((_CHUNK,), jnp.int32),
            pltpu.VMEM((_CHUNK,), jnp.int32),
            pltpu.VMEM((srows, 128), jnp.float32),
            pltpu.VMEM((srows, 128), jnp.float32),
            pltpu.VMEM((_CHUNK, icp), jnp.float32),
            pltpu.VMEM((_CHUNK, icp), jnp.float32),
            pltpu.VMEM_SHARED((npad, icp), jnp.float32),
            pltpu.VMEM((zrows, icp), jnp.float32),
            pltpu.SemaphoreType.DMA,
            pltpu.SemaphoreType.DMA,
            pltpu.SemaphoreType.DMA,
            pltpu.SemaphoreType.DMA,
        ],
        compiler_params=pltpu.CompilerParams(use_tc_tiling_on_sc=False),
    )
    def sc_kernel(xc_hbm, src_hbm, dst_hbm, s_hbm, out_hbm,
                  srcv0, srcv1, dstv0, dstv1, sv0, sv1, xv0, xv1,
                  acc, zbuf, ls0, ls1, gs0, gs1):
        c = lax.axis_index("c")
        s = lax.axis_index("s")
        srcv = (srcv0, srcv1)
        dstv = (dstv0, dstv1)
        sv = (sv0, sv1)
        xv = (xv0, xv1)
        ls = (ls0, ls1)
        gs = (gs0, gs1)

        def zrow(r, _):
            for k in range(nvec):
                zbuf[r, pl.ds(k * _LANES, _LANES)] = jnp.zeros(
                    (_LANES,), jnp.float32)
            return 0
        lax.fori_loop(0, zrows, zrow, 0)

        def zcopy(j, _):
            pltpu.sync_copy(
                zbuf, acc.at[pl.ds(s * rpt + j * zrows, zrows), :])
            return 0
        lax.fori_loop(0, rpt // zrows, zcopy, 0)
        plsc.subcore_barrier()

        def loads_descr(cj, b):
            tr = s * trpt + cj
            return (
                pltpu.make_async_copy(
                    src_hbm.at[pl.ds(tr * _CHUNK, _CHUNK)], srcv[b], ls[b]),
                pltpu.make_async_copy(
                    dst_hbm.at[pl.ds(tr * _CHUNK, _CHUNK)], dstv[b], ls[b]),
                pltpu.make_async_copy(
                    s_hbm.at[pl.ds((c * ntr + tr) * srows, srows), :],
                    sv[b], ls[b]),
            )

        def issue_loads(cj, b):
            for d in loads_descr(cj, b):
                d.start()

        def wait_loads(cj, b):
            for d in loads_descr(cj, b):
                d.wait()

        def gather_descr(b):
            return pltpu.make_async_copy(xc_hbm.at[srcv[b]], xv[b], gs[b])

        def process(cj, b, nb):
            cj1 = jnp.minimum(cj + 1, trpt - 1)
            wait_loads(cj1, nb)
            gather_descr(nb).start()
            gather_descr(b).wait()

            def erow(r, _):
                for d, k, row8, l in dk:
                    slk = pl.ds(k * _LANES, _LANES)
                    sll = pl.ds(l, _LANES)
                    xv[b][8 * r + d, slk] = (
                        xv[b][8 * r + d, slk] * sv[b][row8 + r, sll])
                return 0
            lax.fori_loop(0, _GRP, erow, 0)

            pltpu.sync_copy(xv[b], acc.at[dstv[b]], add=True)

            cj2 = jnp.minimum(cj + 2, trpt - 1)
            issue_loads(cj2, b)

        issue_loads(0, 0)
        wait_loads(0, 0)
        gather_descr(0).start()
        issue_loads(1, 1)
        def pair(j, _):
            process(2 * j, 0, 1)
            process(2 * j + 1, 1, 0)
            return 0
        lax.fori_loop(0, npairs, pair, 0)
        gather_descr(0).wait()
        wait_loads(trpt - 1, 1)
        plsc.subcore_barrier()

        pltpu.sync_copy(
            acc.at[pl.ds(s * rpt, rpt), :],
            out_hbm.at[pl.ds(c * npad + s * rpt, rpt), :])

    return sc_kernel


def kernel(x, edge_index, edge_attr, node_attr, W_in, b_in, W_out, b_out):
    n, d = x.shape
    na = node_attr.shape[1]
    e = edge_index.shape[1]
    ea = edge_attr.shape[1]
    ic = d + na
    oc = W_out.shape[0]
    icp = ((ic + _LANES - 1) // _LANES) * _LANES
    npad = ((n + 1 + _NS * 8 - 1) // (_NS * 8)) * (_NS * 8)
    unit = 2 * _NS * _CHUNK
    e_pad = ((e + unit - 1) // unit) * unit
    ngrp = e_pad // _GRP
    ntr = e_pad // _CHUNK
    blk = _GRP * icp
    nlt = blk // 128

    f32 = jnp.float32
    src = jnp.pad(edge_index[0].astype(jnp.int32), (0, e_pad - e))
    dst = jnp.pad(edge_index[1].astype(jnp.int32), (0, e_pad - e),
                  constant_values=n)

    xc = jnp.concatenate([x.astype(f32), node_attr.astype(f32)], axis=1)
    xcp = jnp.pad(xc, ((0, 0), (0, icp - ic)))

    ea8 = jnp.pad(edge_attr.astype(f32), ((0, e_pad - e), (0, 0))
                  ).reshape(ngrp, _GRP * ea)

    eye8 = jnp.eye(_GRP, dtype=f32)
    w_blk, b_blk = [], []
    for h in range(2):
        w_h = jnp.pad(W_in[h::2, :], ((0, icp - ic), (0, 0))).astype(f32)
        b_h = jnp.pad(b_in[h::2], (0, icp - ic)).astype(f32)
        w_blk.append(jnp.einsum("de,ca->daec", eye8, w_h)
                     .reshape(_GRP * ea, blk))
        b_blk.append(jnp.tile(b_h, _GRP))
    w_blk = jnp.stack(w_blk)
    b_blk = jnp.stack(b_blk)[:, None, :]

    tt = 1256
    nsteps = ngrp // tt
    s_pk = pl.pallas_call(
        functools.partial(_tc_scaling_body, tt=tt),
        grid=(2, nsteps),
        in_specs=[
            pl.BlockSpec((tt, _GRP * ea), lambda h, i: (i, 0)),
            pl.BlockSpec((1, _GRP * ea, blk), lambda h, i: (h, 0, 0)),
            pl.BlockSpec((1, 1, blk), lambda h, i: (h, 0, 0)),
        ],
        out_specs=pl.BlockSpec(
            (1, tt // _GRP, _ROWS * _GRP, 128), lambda h, i: (h, i, 0, 0)),
        out_shape=jax.ShapeDtypeStruct(
            (2, ntr, _ROWS * _GRP, 128), f32),
    )(ea8, w_blk, b_blk)
    s_flat = s_pk.reshape(2 * ntr * _ROWS * _GRP, 128)

    sc_fn = _sc_gather_scale_scatter(ntr, icp, npad)
    aggr = sc_fn(xcp, src, dst, s_flat)
    a0 = aggr[:n]
    a1 = aggr[npad:npad + n]

    w0o = jnp.pad(W_out[:, 0::2], ((0, 0), (0, icp - ic))).T.astype(f32)
    w1o = jnp.pad(W_out[:, 1::2], ((0, 0), (0, icp - ic))).T.astype(f32)
    b_o = b_out[None, :].astype(f32)

    tile_n = 1000
    out = pl.pallas_call(
        _tc_out_body,
        grid=(n // tile_n,),
        in_specs=[
            pl.BlockSpec((tile_n, icp), lambda i: (i, 0)),
            pl.BlockSpec((tile_n, icp), lambda i: (i, 0)),
            pl.BlockSpec((icp, oc), lambda i: (0, 0)),
            pl.BlockSpec((icp, oc), lambda i: (0, 0)),
            pl.BlockSpec((1, oc), lambda i: (0, 0)),
        ],
        out_specs=pl.BlockSpec((tile_n, oc), lambda i: (i, 0)),
        out_shape=jax.ShapeDtypeStruct((n, oc), f32),
    )(a0, a1, w0o, w1o, b_o)
    return out


# speedup vs baseline: 1.5448x; 1.0171x over previous
import functools

import jax
import jax.numpy as jnp
from jax import lax
from jax.experimental import pallas as pl
from jax.experimental.pallas import tpu as pltpu
from jax.experimental.pallas import tpu_sc as plsc

_LANES = 16
_NC = 2
_NS = 16
_CHUNK = 64
_GRP = 8
_ROWS = 9


def _tc_scaling_body(ea8_ref, w_ref, b_ref, out_ref, *, tt):
    ea8 = ea8_ref[...]
    w = w_ref[0]
    b = b_ref[0]
    for L in range(_ROWS):
        m = jnp.dot(ea8, w[:, 128 * L:128 * (L + 1)],
                    preferred_element_type=jnp.float32)
        m = jnp.maximum(m + b[:, 128 * L:128 * (L + 1)], 0.0)
        out_ref[0, :, 8 * L:8 * (L + 1), :] = m.reshape(tt // 8, 8, 128)


def _tc_out_body(a0a_ref, a0b_ref, a1a_ref, a1b_ref,
                 w0_ref, w1_ref, b_ref, out_ref):
    a0 = a0a_ref[...] + a0b_ref[...]
    a1 = a1a_ref[...] + a1b_ref[...]
    acc = jnp.dot(a0, w0_ref[...], preferred_element_type=jnp.float32)
    acc = acc + jnp.dot(a1, w1_ref[...], preferred_element_type=jnp.float32)
    out_ref[...] = jnp.tanh(acc + b_ref[...])


def _sc_gather_scale_scatter(ntr, icp, npad):
    trpt = ntr // _NS
    npairs = trpt // 2
    rpt = npad // _NS
    nvec = icp // _LANES
    zrows = 8
    srows = _ROWS * _GRP

    dk = [(d, k, 8 * ((144 * d + 16 * k) // 128), (144 * d + 16 * k) % 128)
          for d in range(_GRP) for k in range(nvec)]

    mesh = plsc.VectorSubcoreMesh(
        core_axis_name="c", subcore_axis_name="s",
        num_cores=_NC, num_subcores=_NS)

    @functools.partial(
        pl.kernel,
        out_type=jax.ShapeDtypeStruct((_NC * npad, icp), jnp.float32),
        mesh=mesh,
        scratch_types=[
            pltpu.VMEM((_CHUNK,), jnp.int32),
            pltpu.VMEM((_CHUNK,), jnp.int32),
            pltpu.VMEM((_CHUNK,), jnp.int32),
            pltpu.VMEM((_CHUNK,), jnp.int32),
            pltpu.VMEM((srows, 128), jnp.float32),
            pltpu.VMEM((srows, 128), jnp.float32),
            pltpu.VMEM((_CHUNK, icp), jnp.float32),
            pltpu.VMEM((_CHUNK, icp), jnp.float32),
            pltpu.VMEM_SHARED((npad, icp), jnp.float32),
            pltpu.VMEM((zrows, icp), jnp.float32),
            pltpu.SemaphoreType.DMA,
            pltpu.SemaphoreType.DMA,
            pltpu.SemaphoreType.DMA,
            pltpu.SemaphoreType.DMA,
        ],
        compiler_params=pltpu.CompilerParams(use_tc_tiling_on_sc=False),
    )
    def sc_kernel(xc_hbm, src_hbm, dst_hbm, s_hbm, out_hbm,
                  srcv0, srcv1, dstv0, dstv1, sv0, sv1, xv0, xv1,
                  acc, zbuf, ls0, ls1, gs0, gs1):
        c = lax.axis_index("c")
        s = lax.axis_index("s")
        srcv = (srcv0, srcv1)
        dstv = (dstv0, dstv1)
        sv = (sv0, sv1)
        xv = (xv0, xv1)
        ls = (ls0, ls1)
        gs = (gs0, gs1)

        def zrow(r, _):
            for k in range(nvec):
                zbuf[r, pl.ds(k * _LANES, _LANES)] = jnp.zeros(
                    (_LANES,), jnp.float32)
            return 0
        lax.fori_loop(0, zrows, zrow, 0)

        def zcopy(j, _):
            pltpu.sync_copy(
                zbuf, acc.at[pl.ds(s * rpt + j * zrows, zrows), :])
            return 0
        lax.fori_loop(0, rpt // zrows, zcopy, 0)
        plsc.subcore_barrier()

        def loads_descr(cj, b):
            tr = s * trpt + cj
            return (
                pltpu.make_async_copy(
                    src_hbm.at[pl.ds(tr * _CHUNK, _CHUNK)], srcv[b], ls[b]),
                pltpu.make_async_copy(
                    dst_hbm.at[pl.ds(tr * _CHUNK, _CHUNK)], dstv[b], ls[b]),
                pltpu.make_async_copy(
                    s_hbm.at[pl.ds((c * ntr + tr) * srows, srows), :],
                    sv[b], ls[b]),
            )

        def issue_loads(cj, b):
            for d in loads_descr(cj, b):
                d.start()

        def wait_loads(cj, b):
            for d in loads_descr(cj, b):
                d.wait()

        def gather_descr(b):
            return pltpu.make_async_copy(xc_hbm.at[srcv[b]], xv[b], gs[b])

        def process(cj, b, nb):
            cj1 = jnp.minimum(cj + 1, trpt - 1)
            wait_loads(cj1, nb)
            gather_descr(nb).start()
            gather_descr(b).wait()

            def erow(r, _):
                for d, k, row8, l in dk:
                    slk = pl.ds(k * _LANES, _LANES)
                    sll = pl.ds(l, _LANES)
                    xv[b][8 * r + d, slk] = (
                        xv[b][8 * r + d, slk] * sv[b][row8 + r, sll])
                return 0
            lax.fori_loop(0, _GRP, erow, 0)

            pltpu.sync_copy(xv[b], acc.at[dstv[b]], add=True)

            cj2 = jnp.minimum(cj + 2, trpt - 1)
            issue_loads(cj2, b)

        issue_loads(0, 0)
        wait_loads(0, 0)
        gather_descr(0).start()
        issue_loads(1, 1)
        def pair(j, _):
            process(2 * j, 0, 1)
            process(2 * j + 1, 1, 0)
            return 0
        lax.fori_loop(0, npairs, pair, 0)
        gather_descr(0).wait()
        wait_loads(trpt - 1, 1)
        plsc.subcore_barrier()

        pltpu.sync_copy(
            acc.at[pl.ds(s * rpt, rpt), :],
            out_hbm.at[pl.ds(c * npad + s * rpt, rpt), :])

    return sc_kernel


def kernel(x, edge_index, edge_attr, node_attr, W_in, b_in, W_out, b_out):
    n, d = x.shape
    na = node_attr.shape[1]
    e = edge_index.shape[1]
    ea = edge_attr.shape[1]
    ic = d + na
    oc = W_out.shape[0]
    icp = ((ic + _LANES - 1) // _LANES) * _LANES
    npad = ((n + 1 + _NS * 8 - 1) // (_NS * 8)) * (_NS * 8)
    unit = 2 * 2 * _NS * _CHUNK
    e_pad = ((e + unit - 1) // unit) * unit
    eh = e_pad // 2
    ngrp = eh // _GRP
    ntr = eh // _CHUNK
    blk = _GRP * icp

    f32 = jnp.float32
    src = jnp.pad(edge_index[0].astype(jnp.int32), (0, e_pad - e))
    dst = jnp.pad(edge_index[1].astype(jnp.int32), (0, e_pad - e),
                  constant_values=n)

    xc = jnp.concatenate([x.astype(f32), node_attr.astype(f32)], axis=1)
    xcp = jnp.pad(xc, ((0, 0), (0, icp - ic)))

    ea8 = jnp.pad(edge_attr.astype(f32), ((0, e_pad - e), (0, 0))
                  ).reshape(2, ngrp, _GRP * ea)

    eye8 = jnp.eye(_GRP, dtype=f32)
    w_blk, b_blk = [], []
    for h in range(2):
        w_h = jnp.pad(W_in[h::2, :], ((0, icp - ic), (0, 0))).astype(f32)
        b_h = jnp.pad(b_in[h::2], (0, icp - ic)).astype(f32)
        w_blk.append(jnp.einsum("de,ca->daec", eye8, w_h)
                     .reshape(_GRP * ea, blk))
        b_blk.append(jnp.tile(b_h, _GRP))
    w_blk = jnp.stack(w_blk)
    b_blk = jnp.stack(b_blk)[:, None, :]

    tt = 1264
    nsteps = ngrp // tt
    scale_call = pl.pallas_call(
        functools.partial(_tc_scaling_body, tt=tt),
        grid=(2, nsteps),
        in_specs=[
            pl.BlockSpec((tt, _GRP * ea), lambda h, i: (i, 0)),
            pl.BlockSpec((1, _GRP * ea, blk), lambda h, i: (h, 0, 0)),
            pl.BlockSpec((1, 1, blk), lambda h, i: (h, 0, 0)),
        ],
        out_specs=pl.BlockSpec(
            (1, tt // _GRP, _ROWS * _GRP, 128), lambda h, i: (h, i, 0, 0)),
        out_shape=jax.ShapeDtypeStruct(
            (2, ntr, _ROWS * _GRP, 128), f32),
    )
    sc_fn = _sc_gather_scale_scatter(ntr, icp, npad)

    aggr = []
    for half in range(2):
        s_pk = scale_call(ea8[half], w_blk, b_blk)
        s_flat = s_pk.reshape(2 * ntr * _ROWS * _GRP, 128)
        aggr.append(sc_fn(xcp, src[half * eh:(half + 1) * eh],
                          dst[half * eh:(half + 1) * eh], s_flat))
    a0a, a0b = aggr[0][:n], aggr[1][:n]
    a1a, a1b = aggr[0][npad:npad + n], aggr[1][npad:npad + n]

    w0o = jnp.pad(W_out[:, 0::2], ((0, 0), (0, icp - ic))).T.astype(f32)
    w1o = jnp.pad(W_out[:, 1::2], ((0, 0), (0, icp - ic))).T.astype(f32)
    b_o = b_out[None, :].astype(f32)

    tile_n = 1000
    out = pl.pallas_call(
        _tc_out_body,
        grid=(n // tile_n,),
        in_specs=[
            pl.BlockSpec((tile_n, icp), lambda i: (i, 0)),
            pl.BlockSpec((tile_n, icp), lambda i: (i, 0)),
            pl.BlockSpec((tile_n, icp), lambda i: (i, 0)),
            pl.BlockSpec((tile_n, icp), lambda i: (i, 0)),
            pl.BlockSpec((icp, oc), lambda i: (0, 0)),
            pl.BlockSpec((icp, oc), lambda i: (0, 0)),
            pl.BlockSpec((1, oc), lambda i: (0, 0)),
        ],
        out_specs=pl.BlockSpec((tile_n, oc), lambda i: (i, 0)),
        out_shape=jax.ShapeDtypeStruct((n, oc), f32),
    )(a0a, a0b, a1a, a1b, w0o, w1o, b_o)
    return out


# speedup vs baseline: 1.5496x; 1.0031x over previous
import functools

import jax
import jax.numpy as jnp
from jax import lax
from jax.experimental import pallas as pl
from jax.experimental.pallas import tpu as pltpu
from jax.experimental.pallas import tpu_sc as plsc

_LANES = 16
_NC = 2
_NS = 16
_CHUNK = 64
_GRP = 8
_ROWS = 9


def _tc_scaling_body(ea8_ref, w_ref, b_ref, out_ref, *, tt):
    ea8 = ea8_ref[...]
    w = w_ref[0]
    b = b_ref[0]
    for L in range(_ROWS):
        m = jnp.dot(ea8, w[:, 128 * L:128 * (L + 1)],
                    preferred_element_type=jnp.float32)
        m = jnp.maximum(m + b[:, 128 * L:128 * (L + 1)], 0.0)
        out_ref[0, :, 8 * L:8 * (L + 1), :] = m.reshape(tt // 8, 8, 128)


def _tc_out_body(a0a_ref, a0b_ref, a1a_ref, a1b_ref,
                 w0_ref, w1_ref, b_ref, out_ref):
    a0 = a0a_ref[...] + a0b_ref[...]
    a1 = a1a_ref[...] + a1b_ref[...]
    acc = jnp.dot(a0, w0_ref[...], preferred_element_type=jnp.float32)
    acc = acc + jnp.dot(a1, w1_ref[...], preferred_element_type=jnp.float32)
    out_ref[...] = jnp.tanh(acc + b_ref[...])


def _sc_gather_scale_scatter(ntr, icp, npad):
    trpt = ntr // _NS
    npairs = trpt // 2
    rpt = npad // _NS
    nvec = icp // _LANES
    zrows = 8
    srows = _ROWS * _GRP

    dk = [(d, k, 8 * ((144 * d + 16 * k) // 128), (144 * d + 16 * k) % 128)
          for d in range(_GRP) for k in range(nvec)]

    mesh = plsc.VectorSubcoreMesh(
        core_axis_name="c", subcore_axis_name="s",
        num_cores=_NC, num_subcores=_NS)

    @functools.partial(
        pl.kernel,
        out_type=jax.ShapeDtypeStruct((_NC * npad, icp), jnp.float32),
        mesh=mesh,
        scratch_types=[
            pltpu.VMEM((_CHUNK,), jnp.int32),
            pltpu.VMEM((_CHUNK,), jnp.int32),
            pltpu.VMEM((_CHUNK,), jnp.int32),
            pltpu.VMEM((_CHUNK,), jnp.int32),
            pltpu.VMEM((srows, 128), jnp.float32),
            pltpu.VMEM((srows, 128), jnp.float32),
            pltpu.VMEM((_CHUNK, icp), jnp.float32),
            pltpu.VMEM((_CHUNK, icp), jnp.float32),
            pltpu.VMEM_SHARED((npad, icp), jnp.float32),
            pltpu.VMEM((zrows, icp), jnp.float32),
            pltpu.SemaphoreType.DMA,
            pltpu.SemaphoreType.DMA,
            pltpu.SemaphoreType.DMA,
            pltpu.SemaphoreType.DMA,
            pltpu.SemaphoreType.DMA,
        ],
        compiler_params=pltpu.CompilerParams(use_tc_tiling_on_sc=False),
    )
    def sc_kernel(xc_hbm, src_hbm, dst_hbm, s_hbm, out_hbm,
                  srcv0, srcv1, dstv0, dstv1, sv0, sv1, xv0, xv1,
                  acc, zbuf, zs, ls0, ls1, gs0, gs1):
        c = lax.axis_index("c")
        s = lax.axis_index("s")
        srcv = (srcv0, srcv1)
        dstv = (dstv0, dstv1)
        sv = (sv0, sv1)
        xv = (xv0, xv1)
        ls = (ls0, ls1)
        gs = (gs0, gs1)

        def zrow(r, _):
            for k in range(nvec):
                zbuf[r, pl.ds(k * _LANES, _LANES)] = jnp.zeros(
                    (_LANES,), jnp.float32)
            return 0
        lax.fori_loop(0, zrows, zrow, 0)

        ncopies = rpt // zrows
        for base in range(0, ncopies, 8):
            descrs = [
                pltpu.make_async_copy(
                    zbuf,
                    acc.at[pl.ds(s * rpt + (base + t) * zrows, zrows), :],
                    zs)
                for t in range(min(8, ncopies - base))]
            for d_ in descrs:
                d_.start()
            for d_ in descrs:
                d_.wait()
        plsc.subcore_barrier()

        def loads_descr(cj, b):
            tr = s * trpt + cj
            return (
                pltpu.make_async_copy(
                    src_hbm.at[pl.ds(tr * _CHUNK, _CHUNK)], srcv[b], ls[b]),
                pltpu.make_async_copy(
                    dst_hbm.at[pl.ds(tr * _CHUNK, _CHUNK)], dstv[b], ls[b]),
                pltpu.make_async_copy(
                    s_hbm.at[pl.ds((c * ntr + tr) * srows, srows), :],
                    sv[b], ls[b]),
            )

        def issue_loads(cj, b):
            for d in loads_descr(cj, b):
                d.start()

        def wait_loads(cj, b):
            for d in loads_descr(cj, b):
                d.wait()

        def gather_descr(b):
            return pltpu.make_async_copy(xc_hbm.at[srcv[b]], xv[b], gs[b])

        def process(cj, b, nb):
            cj1 = jnp.minimum(cj + 1, trpt - 1)
            wait_loads(cj1, nb)
            gather_descr(nb).start()
            gather_descr(b).wait()

            def erow(r, _):
                for d, k, row8, l in dk:
                    slk = pl.ds(k * _LANES, _LANES)
                    sll = pl.ds(l, _LANES)
                    xv[b][8 * r + d, slk] = (
                        xv[b][8 * r + d, slk] * sv[b][row8 + r, sll])
                return 0
            lax.fori_loop(0, _GRP, erow, 0)

            pltpu.sync_copy(xv[b], acc.at[dstv[b]], add=True)

            cj2 = jnp.minimum(cj + 2, trpt - 1)
            issue_loads(cj2, b)

        issue_loads(0, 0)
        wait_loads(0, 0)
        gather_descr(0).start()
        issue_loads(1, 1)
        def pair(j, _):
            process(2 * j, 0, 1)
            process(2 * j + 1, 1, 0)
            return 0
        lax.fori_loop(0, npairs, pair, 0)
        gather_descr(0).wait()
        wait_loads(trpt - 1, 1)
        plsc.subcore_barrier()

        pltpu.sync_copy(
            acc.at[pl.ds(s * rpt, rpt), :],
            out_hbm.at[pl.ds(c * npad + s * rpt, rpt), :])

    return sc_kernel


def kernel(x, edge_index, edge_attr, node_attr, W_in, b_in, W_out, b_out):
    n, d = x.shape
    na = node_attr.shape[1]
    e = edge_index.shape[1]
    ea = edge_attr.shape[1]
    ic = d + na
    oc = W_out.shape[0]
    icp = ((ic + _LANES - 1) // _LANES) * _LANES
    npad = ((n + 1 + _NS * 8 - 1) // (_NS * 8)) * (_NS * 8)
    unit = 2 * 2 * _NS * _CHUNK
    e_pad = ((e + unit - 1) // unit) * unit
    eh = e_pad // 2
    ngrp = eh // _GRP
    ntr = eh // _CHUNK
    blk = _GRP * icp

    f32 = jnp.float32
    src = jnp.pad(edge_index[0].astype(jnp.int32), (0, e_pad - e))
    dst = jnp.pad(edge_index[1].astype(jnp.int32), (0, e_pad - e),
                  constant_values=n)

    xc = jnp.concatenate([x.astype(f32), node_attr.astype(f32)], axis=1)
    xcp = jnp.pad(xc, ((0, 0), (0, icp - ic)))

    ea8 = jnp.pad(edge_attr.astype(f32), ((0, e_pad - e), (0, 0))
                  ).reshape(2, ngrp, _GRP * ea)

    eye8 = jnp.eye(_GRP, dtype=f32)
    w_blk, b_blk = [], []
    for h in range(2):
        w_h = jnp.pad(W_in[h::2, :], ((0, icp - ic), (0, 0))).astype(f32)
        b_h = jnp.pad(b_in[h::2], (0, icp - ic)).astype(f32)
        w_blk.append(jnp.einsum("de,ca->daec", eye8, w_h)
                     .reshape(_GRP * ea, blk))
        b_blk.append(jnp.tile(b_h, _GRP))
    w_blk = jnp.stack(w_blk)
    b_blk = jnp.stack(b_blk)[:, None, :]

    tt = 1264
    nsteps = ngrp // tt
    scale_call = pl.pallas_call(
        functools.partial(_tc_scaling_body, tt=tt),
        grid=(2, nsteps),
        in_specs=[
            pl.BlockSpec((tt, _GRP * ea), lambda h, i: (i, 0)),
            pl.BlockSpec((1, _GRP * ea, blk), lambda h, i: (h, 0, 0)),
            pl.BlockSpec((1, 1, blk), lambda h, i: (h, 0, 0)),
        ],
        out_specs=pl.BlockSpec(
            (1, tt // _GRP, _ROWS * _GRP, 128), lambda h, i: (h, i, 0, 0)),
        out_shape=jax.ShapeDtypeStruct(
            (2, ntr, _ROWS * _GRP, 128), f32),
    )
    sc_fn = _sc_gather_scale_scatter(ntr, icp, npad)

    aggr = []
    for half in range(2):
        s_pk = scale_call(ea8[half], w_blk, b_blk)
        s_flat = s_pk.reshape(2 * ntr * _ROWS * _GRP, 128)
        aggr.append(sc_fn(xcp, src[half * eh:(half + 1) * eh],
                          dst[half * eh:(half + 1) * eh], s_flat))
    a0a, a0b = aggr[0][:n], aggr[1][:n]
    a1a, a1b = aggr[0][npad:npad + n], aggr[1][npad:npad + n]

    w0o = jnp.pad(W_out[:, 0::2], ((0, 0), (0, icp - ic))).T.astype(f32)
    w1o = jnp.pad(W_out[:, 1::2], ((0, 0), (0, icp - ic))).T.astype(f32)
    b_o = b_out[None, :].astype(f32)

    tile_n = 1000
    out = pl.pallas_call(
        _tc_out_body,
        grid=(n // tile_n,),
        in_specs=[
            pl.BlockSpec((tile_n, icp), lambda i: (i, 0)),
            pl.BlockSpec((tile_n, icp), lambda i: (i, 0)),
            pl.BlockSpec((tile_n, icp), lambda i: (i, 0)),
            pl.BlockSpec((tile_n, icp), lambda i: (i, 0)),
            pl.BlockSpec((icp, oc), lambda i: (0, 0)),
            pl.BlockSpec((icp, oc), lambda i: (0, 0)),
            pl.BlockSpec((1, oc), lambda i: (0, 0)),
        ],
        out_specs=pl.BlockSpec((tile_n, oc), lambda i: (i, 0)),
        out_shape=jax.ShapeDtypeStruct((n, oc), f32),
    )(a0a, a0b, a1a, a1b, w0o, w1o, b_o)
    return out


# speedup vs baseline: 1.6342x; 1.0546x over previous
import functools

import jax
import jax.numpy as jnp
from jax import lax
from jax.experimental import pallas as pl
from jax.experimental.pallas import tpu as pltpu
from jax.experimental.pallas import tpu_sc as plsc

_LANES = 16
_NC = 2
_NS = 16
_CHUNK = 64
_GRP = 8
_ROWS = 9


def _tc_scaling_body(ea8_ref, w_ref, b_ref, out_ref, *, tt):
    ea8 = ea8_ref[...]
    w = w_ref[0]
    b = b_ref[0]
    for L in range(_ROWS):
        m = jnp.dot(ea8, w[:, 128 * L:128 * (L + 1)],
                    preferred_element_type=jnp.float32)
        m = jnp.maximum(m + b[:, 128 * L:128 * (L + 1)], 0.0)
        out_ref[0, :, 8 * L:8 * (L + 1), :] = m.reshape(tt // 8, 8, 128)


def _tc_out_body(a0a_ref, a0b_ref, a1a_ref, a1b_ref,
                 w0_ref, w1_ref, b_ref, out_ref):
    a0 = a0a_ref[...] + a0b_ref[...]
    a1 = a1a_ref[...] + a1b_ref[...]
    acc = jnp.dot(a0, w0_ref[...], preferred_element_type=jnp.float32)
    acc = acc + jnp.dot(a1, w1_ref[...], preferred_element_type=jnp.float32)
    out_ref[...] = jnp.tanh(acc + b_ref[...])


def _sc_gather_scale_scatter(ntr, icp, npad):
    trpt = ntr // _NS
    npairs = trpt // 2
    rpt = npad // _NS
    nvec = icp // _LANES
    zrows = 8
    srows = _ROWS * _GRP

    dk = [(d, k, 8 * ((144 * d + 16 * k) // 128), (144 * d + 16 * k) % 128)
          for d in range(_GRP) for k in range(nvec)]

    mesh = plsc.VectorSubcoreMesh(
        core_axis_name="c", subcore_axis_name="s",
        num_cores=_NC, num_subcores=_NS)

    @functools.partial(
        pl.kernel,
        out_type=jax.ShapeDtypeStruct((_NC * npad, icp), jnp.float32),
        mesh=mesh,
        scratch_types=[
            pltpu.VMEM((_CHUNK,), jnp.int32),
            pltpu.VMEM((_CHUNK,), jnp.int32),
            pltpu.VMEM((_CHUNK,), jnp.int32),
            pltpu.VMEM((_CHUNK,), jnp.int32),
            pltpu.VMEM((_CHUNK,), jnp.int32),
            pltpu.VMEM((_CHUNK,), jnp.int32),
            pltpu.VMEM((srows, 128), jnp.float32),
            pltpu.VMEM((srows, 128), jnp.float32),
            pltpu.VMEM((_CHUNK, icp), jnp.float32),
            pltpu.VMEM((_CHUNK, icp), jnp.float32),
            pltpu.VMEM_SHARED((npad, icp), jnp.float32),
            pltpu.VMEM((zrows, icp), jnp.float32),
            pltpu.SemaphoreType.DMA,
            pltpu.SemaphoreType.DMA,
            pltpu.SemaphoreType.DMA,
            pltpu.SemaphoreType.DMA,
            pltpu.SemaphoreType.DMA,
            pltpu.SemaphoreType.DMA,
            pltpu.SemaphoreType.DMA,
        ],
        compiler_params=pltpu.CompilerParams(use_tc_tiling_on_sc=False),
    )
    def sc_kernel(xc_hbm, src_hbm, dst_hbm, s_hbm, out_hbm,
                  srcv0, srcv1, dstv0, dstv1, dsts0, dsts1, sv0, sv1,
                  xv0, xv1, acc, zbuf, zs, ls0, ls1, gs0, gs1, ss0, ss1):
        c = lax.axis_index("c")
        s = lax.axis_index("s")
        srcv = (srcv0, srcv1)
        dstv = (dstv0, dstv1)
        dsts = (dsts0, dsts1)
        sv = (sv0, sv1)
        xv = (xv0, xv1)
        ls = (ls0, ls1)
        gs = (gs0, gs1)
        ss = (ss0, ss1)

        def zrow(r, _):
            for k in range(nvec):
                zbuf[r, pl.ds(k * _LANES, _LANES)] = jnp.zeros(
                    (_LANES,), jnp.float32)
            return 0
        lax.fori_loop(0, zrows, zrow, 0)

        ncopies = rpt // zrows
        for base in range(0, ncopies, 8):
            descrs = [
                pltpu.make_async_copy(
                    zbuf,
                    acc.at[pl.ds(s * rpt + (base + t) * zrows, zrows), :],
                    zs)
                for t in range(min(8, ncopies - base))]
            for d_ in descrs:
                d_.start()
            for d_ in descrs:
                d_.wait()
        plsc.subcore_barrier()

        def loads_descr(cj, b):
            tr = s * trpt + cj
            return (
                pltpu.make_async_copy(
                    src_hbm.at[pl.ds(tr * _CHUNK, _CHUNK)], srcv[b], ls[b]),
                pltpu.make_async_copy(
                    dst_hbm.at[pl.ds(tr * _CHUNK, _CHUNK)], dstv[b], ls[b]),
                pltpu.make_async_copy(
                    s_hbm.at[pl.ds((c * ntr + tr) * srows, srows), :],
                    sv[b], ls[b]),
            )

        def issue_loads(cj, b):
            for d in loads_descr(cj, b):
                d.start()

        def wait_loads(cj, b):
            for d in loads_descr(cj, b):
                d.wait()

        def gather_descr(b):
            return pltpu.make_async_copy(xc_hbm.at[srcv[b]], xv[b], gs[b])

        def sct_start(b):
            pltpu.async_copy(xv[b], acc.at[dsts[b]], ss[b], add=True)

        def sct_wait(b):
            pltpu.make_async_copy(xv[b], acc.at[dsts[b]], ss[b]).wait()

        def multiply(b):
            def erow(r, _):
                for d, k, row8, l in dk:
                    slk = pl.ds(k * _LANES, _LANES)
                    sll = pl.ds(l, _LANES)
                    xv[b][8 * r + d, slk] = (
                        xv[b][8 * r + d, slk] * sv[b][row8 + r, sll])
                return 0
            lax.fori_loop(0, _GRP, erow, 0)
            for q in range(_CHUNK // _LANES):
                sl = pl.ds(q * _LANES, _LANES)
                dsts[b][sl] = dstv[b][sl]

        def process(cj, b, nb, first=False):
            cj1 = jnp.minimum(cj + 1, trpt - 1)
            wait_loads(cj1, nb)
            if not first:
                sct_wait(nb)
            gather_descr(nb).start()
            gather_descr(b).wait()
            multiply(b)
            sct_start(b)
            cj2 = jnp.minimum(cj + 2, trpt - 1)
            issue_loads(cj2, b)

        issue_loads(0, 0)
        wait_loads(0, 0)
        gather_descr(0).start()
        issue_loads(1, 1)
        process(0, 0, 1, first=True)
        def pair(j, _):
            process(2 * j + 1, 1, 0)
            process(2 * j + 2, 0, 1)
            return 0
        lax.fori_loop(0, (trpt - 2) // 2, pair, 0)
        process(trpt - 1, 1, 0)
        sct_wait(1)
        gather_descr(0).wait()
        wait_loads(trpt - 1, 1)
        plsc.subcore_barrier()

        pltpu.sync_copy(
            acc.at[pl.ds(s * rpt, rpt), :],
            out_hbm.at[pl.ds(c * npad + s * rpt, rpt), :])

    return sc_kernel


def kernel(x, edge_index, edge_attr, node_attr, W_in, b_in, W_out, b_out):
    n, d = x.shape
    na = node_attr.shape[1]
    e = edge_index.shape[1]
    ea = edge_attr.shape[1]
    ic = d + na
    oc = W_out.shape[0]
    icp = ((ic + _LANES - 1) // _LANES) * _LANES
    npad = ((n + 1 + _NS * 8 - 1) // (_NS * 8)) * (_NS * 8)
    unit = 2 * 2 * _NS * _CHUNK
    e_pad = ((e + unit - 1) // unit) * unit
    eh = e_pad // 2
    ngrp = eh // _GRP
    ntr = eh // _CHUNK
    blk = _GRP * icp

    f32 = jnp.float32
    src = jnp.pad(edge_index[0].astype(jnp.int32), (0, e_pad - e))
    dst = jnp.pad(edge_index[1].astype(jnp.int32), (0, e_pad - e),
                  constant_values=n)

    xc = jnp.concatenate([x.astype(f32), node_attr.astype(f32)], axis=1)
    xcp = jnp.pad(xc, ((0, 0), (0, icp - ic)))

    ea8 = jnp.pad(edge_attr.astype(f32), ((0, e_pad - e), (0, 0))
                  ).reshape(2, ngrp, _GRP * ea)

    eye8 = jnp.eye(_GRP, dtype=f32)
    w_blk, b_blk = [], []
    for h in range(2):
        w_h = jnp.pad(W_in[h::2, :], ((0, icp - ic), (0, 0))).astype(f32)
        b_h = jnp.pad(b_in[h::2], (0, icp - ic)).astype(f32)
        w_blk.append(jnp.einsum("de,ca->daec", eye8, w_h)
                     .reshape(_GRP * ea, blk))
        b_blk.append(jnp.tile(b_h, _GRP))
    w_blk = jnp.stack(w_blk)
    b_blk = jnp.stack(b_blk)[:, None, :]

    tt = 1264
    nsteps = ngrp // tt
    scale_call = pl.pallas_call(
        functools.partial(_tc_scaling_body, tt=tt),
        grid=(2, nsteps),
        in_specs=[
            pl.BlockSpec((tt, _GRP * ea), lambda h, i: (i, 0)),
            pl.BlockSpec((1, _GRP * ea, blk), lambda h, i: (h, 0, 0)),
            pl.BlockSpec((1, 1, blk), lambda h, i: (h, 0, 0)),
        ],
        out_specs=pl.BlockSpec(
            (1, tt // _GRP, _ROWS * _GRP, 128), lambda h, i: (h, i, 0, 0)),
        out_shape=jax.ShapeDtypeStruct(
            (2, ntr, _ROWS * _GRP, 128), f32),
    )
    sc_fn = _sc_gather_scale_scatter(ntr, icp, npad)

    aggr = []
    for half in range(2):
        s_pk = scale_call(ea8[half], w_blk, b_blk)
        s_flat = s_pk.reshape(2 * ntr * _ROWS * _GRP, 128)
        aggr.append(sc_fn(xcp, src[half * eh:(half + 1) * eh],
                          dst[half * eh:(half + 1) * eh], s_flat))
    a0a, a0b = aggr[0][:n], aggr[1][:n]
    a1a, a1b = aggr[0][npad:npad + n], aggr[1][npad:npad + n]

    w0o = jnp.pad(W_out[:, 0::2], ((0, 0), (0, icp - ic))).T.astype(f32)
    w1o = jnp.pad(W_out[:, 1::2], ((0, 0), (0, icp - ic))).T.astype(f32)
    b_o = b_out[None, :].astype(f32)

    tile_n = 1000
    out = pl.pallas_call(
        _tc_out_body,
        grid=(n // tile_n,),
        in_specs=[
            pl.BlockSpec((tile_n, icp), lambda i: (i, 0)),
            pl.BlockSpec((tile_n, icp), lambda i: (i, 0)),
            pl.BlockSpec((tile_n, icp), lambda i: (i, 0)),
            pl.BlockSpec((tile_n, icp), lambda i: (i, 0)),
            pl.BlockSpec((icp, oc), lambda i: (0, 0)),
            pl.BlockSpec((icp, oc), lambda i: (0, 0)),
            pl.BlockSpec((1, oc), lambda i: (0, 0)),
        ],
        out_specs=pl.BlockSpec((tile_n, oc), lambda i: (i, 0)),
        out_shape=jax.ShapeDtypeStruct((n, oc), f32),
    )(a0a, a0b, a1a, a1b, w0o, w1o, b_o)
    return out


# speedup vs baseline: 1.7948x; 1.0982x over previous
import functools

import jax
import jax.numpy as jnp
from jax import lax
from jax.experimental import pallas as pl
from jax.experimental.pallas import tpu as pltpu
from jax.experimental.pallas import tpu_sc as plsc

_LANES = 16
_NC = 2
_NS = 16
_CHUNK = 64
_GRP = 8
_ROWS = 9


def _tc_scaling_body(ea8_ref, w_ref, b_ref, out_ref, *, tt):
    ea8 = ea8_ref[...]
    w = w_ref[0]
    b = b_ref[0]
    for L in range(_ROWS):
        m = jnp.dot(ea8, w[:, 128 * L:128 * (L + 1)],
                    preferred_element_type=jnp.float32)
        m = jnp.maximum(m + b[:, 128 * L:128 * (L + 1)], 0.0)
        out_ref[0, :, 8 * L:8 * (L + 1), :] = m.reshape(tt // 8, 8, 128)


def _tc_out_body(a0a_ref, a0b_ref, a1a_ref, a1b_ref,
                 w0_ref, w1_ref, b_ref, out_ref):
    a0 = a0a_ref[...] + a0b_ref[...]
    a1 = a1a_ref[...] + a1b_ref[...]
    acc = jnp.dot(a0, w0_ref[...], preferred_element_type=jnp.float32)
    acc = acc + jnp.dot(a1, w1_ref[...], preferred_element_type=jnp.float32)
    out_ref[...] = jnp.tanh(acc + b_ref[...])


def _sc_gather_scale_scatter(ntr, icp, npad):
    trpt = ntr // _NS
    npairs = trpt // 2
    rpt = npad // _NS
    nvec = icp // _LANES
    zrows = 8
    srows = _ROWS * _GRP

    dk = [(d, k, 8 * ((144 * d + 16 * k) // 128), (144 * d + 16 * k) % 128)
          for d in range(_GRP) for k in range(nvec)]

    mesh = plsc.VectorSubcoreMesh(
        core_axis_name="c", subcore_axis_name="s",
        num_cores=_NC, num_subcores=_NS)

    @functools.partial(
        pl.kernel,
        out_type=jax.ShapeDtypeStruct((_NC * npad, icp), jnp.float32),
        mesh=mesh,
        scratch_types=[
            pltpu.VMEM((_CHUNK,), jnp.int32),
            pltpu.VMEM((_CHUNK,), jnp.int32),
            pltpu.VMEM((_CHUNK,), jnp.int32),
            pltpu.VMEM((_CHUNK,), jnp.int32),
            pltpu.VMEM((_CHUNK,), jnp.int32),
            pltpu.VMEM((_CHUNK,), jnp.int32),
            pltpu.VMEM((srows, 128), jnp.float32),
            pltpu.VMEM((srows, 128), jnp.float32),
            pltpu.VMEM((_CHUNK, icp), jnp.float32),
            pltpu.VMEM((_CHUNK, icp), jnp.float32),
            pltpu.VMEM_SHARED((npad, icp), jnp.float32),
            pltpu.VMEM((zrows, icp), jnp.float32),
            pltpu.SemaphoreType.DMA,
            pltpu.SemaphoreType.DMA,
            pltpu.SemaphoreType.DMA,
            pltpu.SemaphoreType.DMA,
            pltpu.SemaphoreType.DMA,
            pltpu.SemaphoreType.DMA,
            pltpu.SemaphoreType.DMA,
        ],
        compiler_params=pltpu.CompilerParams(use_tc_tiling_on_sc=False),
    )
    def sc_kernel(xc_hbm, src_hbm, dst_hbm, s_hbm, out_hbm,
                  srcv0, srcv1, dstv0, dstv1, dsts0, dsts1, sv0, sv1,
                  xv0, xv1, acc, zbuf, zs, ls0, ls1, gs0, gs1, ss0, ss1):
        c = lax.axis_index("c")
        s = lax.axis_index("s")
        srcv = (srcv0, srcv1)
        dstv = (dstv0, dstv1)
        dsts = (dsts0, dsts1)
        sv = (sv0, sv1)
        xv = (xv0, xv1)
        ls = (ls0, ls1)
        gs = (gs0, gs1)
        ss = (ss0, ss1)

        def zrow(r, _):
            for k in range(nvec):
                zbuf[r, pl.ds(k * _LANES, _LANES)] = jnp.zeros(
                    (_LANES,), jnp.float32)
            return 0
        lax.fori_loop(0, zrows, zrow, 0)

        ncopies = rpt // zrows
        for base in range(0, ncopies, 8):
            descrs = [
                pltpu.make_async_copy(
                    zbuf,
                    acc.at[pl.ds(s * rpt + (base + t) * zrows, zrows), :],
                    zs)
                for t in range(min(8, ncopies - base))]
            for d_ in descrs:
                d_.start()
            for d_ in descrs:
                d_.wait()
        plsc.subcore_barrier()

        def loads_descr(cj, b):
            tr = s * trpt + cj
            return (
                pltpu.make_async_copy(
                    src_hbm.at[pl.ds(tr * _CHUNK, _CHUNK)], srcv[b], ls[b]),
                pltpu.make_async_copy(
                    dst_hbm.at[pl.ds(tr * _CHUNK, _CHUNK)], dstv[b], ls[b]),
                pltpu.make_async_copy(
                    s_hbm.at[pl.ds((c * ntr + tr) * srows, srows), :],
                    sv[b], ls[b]),
            )

        def issue_loads(cj, b):
            for d in loads_descr(cj, b):
                d.start()

        def wait_loads(cj, b):
            for d in loads_descr(cj, b):
                d.wait()

        def gather_descr(b):
            return pltpu.make_async_copy(xc_hbm.at[srcv[b]], xv[b], gs[b])

        def sct_start(b):
            pltpu.async_copy(xv[b], acc.at[dsts[b]], ss[b], add=True)

        def sct_wait(b):
            pltpu.make_async_copy(xv[b], acc.at[dsts[b]], ss[b]).wait()

        def multiply(b):
            def erow(r, _):
                for d, k, row8, l in dk:
                    slk = pl.ds(k * _LANES, _LANES)
                    sll = pl.ds(l, _LANES)
                    xv[b][8 * r + d, slk] = (
                        xv[b][8 * r + d, slk] * sv[b][row8 + r, sll])
                return 0
            lax.fori_loop(0, _GRP, erow, 0)
            for q in range(_CHUNK // _LANES):
                sl = pl.ds(q * _LANES, _LANES)
                dsts[b][sl] = dstv[b][sl]

        def process(cj, b, nb, first=False):
            cj1 = jnp.minimum(cj + 1, trpt - 1)
            wait_loads(cj1, nb)
            if not first:
                sct_wait(nb)
            gather_descr(nb).start()
            gather_descr(b).wait()
            multiply(b)
            sct_start(b)
            cj2 = jnp.minimum(cj + 2, trpt - 1)
            issue_loads(cj2, b)

        issue_loads(0, 0)
        wait_loads(0, 0)
        gather_descr(0).start()
        issue_loads(1, 1)
        process(0, 0, 1, first=True)
        def pair(j, _):
            process(2 * j + 1, 1, 0)
            process(2 * j + 2, 0, 1)
            return 0
        lax.fori_loop(0, (trpt - 2) // 2, pair, 0)
        process(trpt - 1, 1, 0)
        sct_wait(1)
        gather_descr(0).wait()
        wait_loads(trpt - 1, 1)
        plsc.subcore_barrier()

        pltpu.sync_copy(
            acc.at[pl.ds(s * rpt, rpt), :],
            out_hbm.at[pl.ds(c * npad + s * rpt, rpt), :])

    return sc_kernel


def kernel(x, edge_index, edge_attr, node_attr, W_in, b_in, W_out, b_out):
    n, d = x.shape
    na = node_attr.shape[1]
    e = edge_index.shape[1]
    ea = edge_attr.shape[1]
    ic = d + na
    oc = W_out.shape[0]
    icp = ((ic + _LANES - 1) // _LANES) * _LANES
    npad = ((n + 1 + _NS * 8 - 1) // (_NS * 8)) * (_NS * 8)
    unit = 2 * _NS * _CHUNK
    e_pad = ((e + unit - 1) // unit) * unit
    units_a = 39
    e_a = units_a * unit
    e_b = e_pad - e_a
    parts = [(0, e_a), (e_a, e_b)]
    blk = _GRP * icp

    f32 = jnp.float32
    src = jnp.pad(edge_index[0].astype(jnp.int32), (0, e_pad - e))
    dst = jnp.pad(edge_index[1].astype(jnp.int32), (0, e_pad - e),
                  constant_values=n)

    xc = jnp.concatenate([x.astype(f32), node_attr.astype(f32)], axis=1)
    xcp = jnp.pad(xc, ((0, 0), (0, icp - ic)))

    ea8 = jnp.pad(edge_attr.astype(f32), ((0, e_pad - e), (0, 0))
                  ).reshape(e_pad // _GRP, _GRP * ea)

    eye8 = jnp.eye(_GRP, dtype=f32)
    w_blk, b_blk = [], []
    for h in range(2):
        w_h = jnp.pad(W_in[h::2, :], ((0, icp - ic), (0, 0))).astype(f32)
        b_h = jnp.pad(b_in[h::2], (0, icp - ic)).astype(f32)
        w_blk.append(jnp.einsum("de,ca->daec", eye8, w_h)
                     .reshape(_GRP * ea, blk))
        b_blk.append(jnp.tile(b_h, _GRP))
    w_blk = jnp.stack(w_blk)
    b_blk = jnp.stack(b_blk)[:, None, :]

    def pick_tt(ngrp_p):
        for cand in range(min(ngrp_p, 1280), 7, -8):
            if ngrp_p % cand == 0:
                return cand
        return _GRP

    aggr = []
    for off, ecnt in parts:
        ngrp_p = ecnt // _GRP
        ntr_p = ecnt // _CHUNK
        tt = pick_tt(ngrp_p)
        s_pk = pl.pallas_call(
            functools.partial(_tc_scaling_body, tt=tt),
            grid=(2, ngrp_p // tt),
            in_specs=[
                pl.BlockSpec((tt, _GRP * ea), lambda h, i: (i, 0)),
                pl.BlockSpec((1, _GRP * ea, blk), lambda h, i: (h, 0, 0)),
                pl.BlockSpec((1, 1, blk), lambda h, i: (h, 0, 0)),
            ],
            out_specs=pl.BlockSpec(
                (1, tt // _GRP, _ROWS * _GRP, 128), lambda h, i: (h, i, 0, 0)),
            out_shape=jax.ShapeDtypeStruct(
                (2, ntr_p, _ROWS * _GRP, 128), f32),
        )(ea8[off // _GRP:(off + ecnt) // _GRP], w_blk, b_blk)
        s_flat = s_pk.reshape(2 * ntr_p * _ROWS * _GRP, 128)
        sc_fn = _sc_gather_scale_scatter(ntr_p, icp, npad)
        aggr.append(sc_fn(xcp, src[off:off + ecnt],
                          dst[off:off + ecnt], s_flat))
    a0a, a0b = aggr[0][:n], aggr[1][:n]
    a1a, a1b = aggr[0][npad:npad + n], aggr[1][npad:npad + n]

    w0o = jnp.pad(W_out[:, 0::2], ((0, 0), (0, icp - ic))).T.astype(f32)
    w1o = jnp.pad(W_out[:, 1::2], ((0, 0), (0, icp - ic))).T.astype(f32)
    b_o = b_out[None, :].astype(f32)

    tile_n = 1000
    out = pl.pallas_call(
        _tc_out_body,
        grid=(n // tile_n,),
        in_specs=[
            pl.BlockSpec((tile_n, icp), lambda i: (i, 0)),
            pl.BlockSpec((tile_n, icp), lambda i: (i, 0)),
            pl.BlockSpec((tile_n, icp), lambda i: (i, 0)),
            pl.BlockSpec((tile_n, icp), lambda i: (i, 0)),
            pl.BlockSpec((icp, oc), lambda i: (0, 0)),
            pl.BlockSpec((icp, oc), lambda i: (0, 0)),
            pl.BlockSpec((1, oc), lambda i: (0, 0)),
        ],
        out_specs=pl.BlockSpec((tile_n, oc), lambda i: (i, 0)),
        out_shape=jax.ShapeDtypeStruct((n, oc), f32),
    )(a0a, a0b, a1a, a1b, w0o, w1o, b_o)
    return out
